# Initial kernel scaffold; baseline (speedup 1.0000x reference)
import jax
import jax.numpy as jnp
from jax.experimental import pallas as pl


def kernel(y_pred, y):
    raise NotImplementedError("write your pallas kernel here")


# speedup vs baseline: 16.4561x; 16.4561x over previous
import functools

import jax
import jax.numpy as jnp
from jax.experimental import pallas as pl
from jax.experimental.pallas import tpu as pltpu

_W_POS = 15.0
_W_NEG = 1.0
_W_KWORST = 5.0

_B, _C, _H, _W = 8, 11, 384, 384
_ROWS_PER_STEP = 128
_STEPS_PER_BATCH = _H // _ROWS_PER_STEP
_GRID = _B * _STEPS_PER_BATCH
_N_PIX = _B * _H * _W
_BLK_PIX = _ROWS_PER_STEP * _W
_SCR_ROWS = _N_PIX // 128
_BLK_ROWS = _BLK_PIX // 128


def _loss_kernel(yp_ref, y_ref, out_ref, bits_ref, acc_ref):
    step = pl.program_id(0)

    @pl.when(step == 0)
    def _init():
        acc_ref[0] = 0.0
        acc_ref[1] = 0.0
        acc_ref[2] = 0.0
        acc_ref[3] = 0.0

    yp = yp_ref[0]
    yv = y_ref[0]

    logit = yp[0]
    mask = (yv[0] != 1).astype(jnp.float32)
    bce = (jnp.maximum(logit, 0.0) - logit * mask
           + jnp.log1p(jnp.exp(-jnp.abs(logit))))
    neg = bce * (1.0 - mask)

    cls_logits = yp[1:]
    tgt = yv[1:].astype(jnp.float32)
    mx = jnp.max(cls_logits, axis=0)
    lse = mx + jnp.log(jnp.sum(jnp.exp(cls_logits - mx), axis=0))
    cls_pix = jnp.sum(tgt, axis=0) * lse - jnp.sum(tgt * cls_logits, axis=0)

    acc_ref[0] += jnp.sum(mask)
    acc_ref[1] += jnp.sum(bce * mask)
    acc_ref[2] += jnp.sum(neg)
    acc_ref[3] += jnp.sum(cls_pix * mask)

    bits_ref[pl.ds(step * _BLK_ROWS, _BLK_ROWS), :] = (
        jax.lax.bitcast_convert_type(neg, jnp.int32).reshape(_BLK_ROWS, 128))

    @pl.when(step == _GRID - 1)
    def _finalize():
        n_pos = acc_ref[0]
        pos_sum = acc_ref[1]
        neg_sum = acc_ref[2]
        cls_sum = acc_ref[3]

        n_pos_i = n_pos.astype(jnp.int32)
        k = jnp.maximum(n_pos_i, 1)
        n_neg = jnp.maximum(_N_PIX - n_pos_i, 1)

        def body(i, prefix):
            cand = prefix | (jnp.int32(1) << (30 - i))
            cnt = jnp.sum((bits_ref[...] >= cand).astype(jnp.int32))
            return jnp.where(cnt >= k, cand, prefix)

        prefix = jax.lax.fori_loop(0, 31, body, jnp.int32(0))

        bits = bits_ref[...]
        vals = jax.lax.bitcast_convert_type(bits, jnp.float32)
        gt = bits > prefix
        cnt_gt = jnp.sum(gt.astype(jnp.int32))
        sum_gt = jnp.sum(jnp.where(gt, vals, 0.0))
        tstar = jax.lax.bitcast_convert_type(prefix, jnp.float32)
        kworst = sum_gt + (k - cnt_gt).astype(jnp.float32) * tstar

        kf = k.astype(jnp.float32)
        nf = n_neg.astype(jnp.float32)
        detection = (_W_POS * pos_sum / kf
                     + _W_NEG * neg_sum / nf
                     + _W_KWORST * kworst / kf)
        out_ref[0, 0] = detection + cls_sum / kf


@functools.partial(jax.jit, static_argnames=("interpret",))
def kernel(y_pred, y, interpret=False):
    out = pl.pallas_call(
        _loss_kernel,
        grid=(_GRID,),
        in_specs=[
            pl.BlockSpec((1, _C, _ROWS_PER_STEP, _W),
                         lambda i: (i // _STEPS_PER_BATCH, 0,
                                    i % _STEPS_PER_BATCH, 0)),
            pl.BlockSpec((1, _C, _ROWS_PER_STEP, _W),
                         lambda i: (i // _STEPS_PER_BATCH, 0,
                                    i % _STEPS_PER_BATCH, 0)),
        ],
        out_specs=pl.BlockSpec(memory_space=pltpu.SMEM),
        out_shape=jax.ShapeDtypeStruct((1, 1), jnp.float32),
        scratch_shapes=[
            pltpu.VMEM((_SCR_ROWS, 128), jnp.int32),
            pltpu.SMEM((8,), jnp.float32),
        ],
        interpret=interpret,
    )(y_pred, y)
    return out[0, 0]


# speedup vs baseline: 30.9456x; 1.8805x over previous
import functools

import jax
import jax.numpy as jnp
from jax.experimental import pallas as pl
from jax.experimental.pallas import tpu as pltpu

_W_POS = 15.0
_W_NEG = 1.0
_W_KWORST = 5.0

_B, _C, _H, _W = 8, 11, 384, 384
_ROWS_PER_STEP = 128
_STEPS_PER_BATCH = _H // _ROWS_PER_STEP
_GRID = _B * _STEPS_PER_BATCH
_N_PIX = _B * _H * _W
_BLK_PIX = _ROWS_PER_STEP * _W
_SCR_ROWS = _N_PIX // 128
_BLK_ROWS = _BLK_PIX // 128


def _loss_kernel(yp_ref, y_ref, out_ref, bits_ref, acc_ref):
    step = pl.program_id(0)

    @pl.when(step == 0)
    def _init():
        acc_ref[0] = 0.0
        acc_ref[1] = 0.0
        acc_ref[2] = 0.0
        acc_ref[3] = 0.0

    yp = yp_ref[0]
    yv = y_ref[0]

    logit = yp[0]
    mask = (yv[0] != 1).astype(jnp.float32)
    bce = (jnp.maximum(logit, 0.0) - logit * mask
           + jnp.log1p(jnp.exp(-jnp.abs(logit))))
    neg = bce * (1.0 - mask)

    cls_logits = yp[1:]
    tgt = yv[1:].astype(jnp.float32)
    mx = jnp.max(cls_logits, axis=0)
    lse = mx + jnp.log(jnp.sum(jnp.exp(cls_logits - mx), axis=0))
    cls_pix = jnp.sum(tgt, axis=0) * lse - jnp.sum(tgt * cls_logits, axis=0)

    acc_ref[0] += jnp.sum(mask)
    acc_ref[1] += jnp.sum(bce * mask)
    acc_ref[2] += jnp.sum(neg)
    acc_ref[3] += jnp.sum(cls_pix * mask)

    bits_ref[pl.ds(step * _BLK_ROWS, _BLK_ROWS), :] = (
        jax.lax.bitcast_convert_type(neg, jnp.int32).reshape(_BLK_ROWS, 128))

    @pl.when(step == _GRID - 1)
    def _finalize():
        n_pos = acc_ref[0]
        pos_sum = acc_ref[1]
        neg_sum = acc_ref[2]
        cls_sum = acc_ref[3]

        n_pos_i = n_pos.astype(jnp.int32)
        k = jnp.maximum(n_pos_i, 1)
        n_neg = jnp.maximum(_N_PIX - n_pos_i, 1)

        easy = k >= (_N_PIX - n_pos_i)

        @pl.when(easy)
        def _all():
            acc_ref[4] = neg_sum

        @pl.when(jnp.logical_not(easy))
        def _select():
            def body(i, prefix):
                cand = prefix | (jnp.int32(1) << (30 - i))
                cnt = jnp.sum((bits_ref[...] >= cand).astype(jnp.int32))
                return jnp.where(cnt >= k, cand, prefix)

            prefix = jax.lax.fori_loop(0, 31, body, jnp.int32(0))

            bits = bits_ref[...]
            vals = jax.lax.bitcast_convert_type(bits, jnp.float32)
            gt = bits > prefix
            cnt_gt = jnp.sum(gt.astype(jnp.int32))
            sum_gt = jnp.sum(jnp.where(gt, vals, 0.0))
            tstar = jax.lax.bitcast_convert_type(prefix, jnp.float32)
            acc_ref[4] = sum_gt + (k - cnt_gt).astype(jnp.float32) * tstar

        kworst = acc_ref[4]

        kf = k.astype(jnp.float32)
        nf = n_neg.astype(jnp.float32)
        detection = (_W_POS * pos_sum / kf
                     + _W_NEG * neg_sum / nf
                     + _W_KWORST * kworst / kf)
        out_ref[0, 0] = detection + cls_sum / kf


@functools.partial(jax.jit, static_argnames=("interpret",))
def kernel(y_pred, y, interpret=False):
    out = pl.pallas_call(
        _loss_kernel,
        grid=(_GRID,),
        in_specs=[
            pl.BlockSpec((1, _C, _ROWS_PER_STEP, _W),
                         lambda i: (i // _STEPS_PER_BATCH, 0,
                                    i % _STEPS_PER_BATCH, 0)),
            pl.BlockSpec((1, _C, _ROWS_PER_STEP, _W),
                         lambda i: (i // _STEPS_PER_BATCH, 0,
                                    i % _STEPS_PER_BATCH, 0)),
        ],
        out_specs=pl.BlockSpec(memory_space=pltpu.SMEM),
        out_shape=jax.ShapeDtypeStruct((1, 1), jnp.float32),
        scratch_shapes=[
            pltpu.VMEM((_SCR_ROWS, 128), jnp.int32),
            pltpu.SMEM((8,), jnp.float32),
        ],
        interpret=interpret,
    )(y_pred, y)
    return out[0, 0]


# speedup vs baseline: 34.8527x; 1.1263x over previous
import functools

import jax
import jax.numpy as jnp
from jax.experimental import pallas as pl
from jax.experimental.pallas import tpu as pltpu

_W_POS = 15.0
_W_NEG = 1.0
_W_KWORST = 5.0

_B, _C, _H, _W = 8, 11, 384, 384
_ROWS_PER_STEP = 192
_STEPS_PER_BATCH = _H // _ROWS_PER_STEP
_GRID = _B * _STEPS_PER_BATCH
_N_PIX = _B * _H * _W
_SCR_ROWS = _GRID * _ROWS_PER_STEP


def _loss_kernel(yp_ref, y_ref, out_ref, bits_ref, acc_ref):
    step = pl.program_id(0)

    @pl.when(step == 0)
    def _init():
        acc_ref[0] = 0.0
        acc_ref[1] = 0.0
        acc_ref[2] = 0.0
        acc_ref[3] = 0.0

    yp = yp_ref[0]
    yv = y_ref[0]

    logit = yp[0]
    mask = (yv[0] != 1).astype(jnp.float32)
    bce = (jnp.maximum(logit, 0.0) - logit * mask
           + jnp.log1p(jnp.exp(-jnp.abs(logit))))
    neg = bce * (1.0 - mask)

    cls_logits = yp[1:]
    tgt_i = yv[1:]
    mx = jnp.max(cls_logits, axis=0)
    lse = mx + jnp.log(jnp.sum(jnp.exp(cls_logits - mx), axis=0))
    t_sum = jnp.sum(tgt_i, axis=0).astype(jnp.float32)
    tx_sum = jnp.sum(jnp.where(tgt_i == 1, cls_logits, 0.0), axis=0)
    cls_pix = t_sum * lse - tx_sum

    acc_ref[0] += jnp.sum(mask)
    acc_ref[1] += jnp.sum(bce * mask)
    acc_ref[2] += jnp.sum(bce)
    acc_ref[3] += jnp.sum(cls_pix * mask)

    bits_ref[pl.ds(step * _ROWS_PER_STEP, _ROWS_PER_STEP), :] = (
        jax.lax.bitcast_convert_type(neg, jnp.int32))

    @pl.when(step == _GRID - 1)
    def _finalize():
        n_pos = acc_ref[0]
        pos_sum = acc_ref[1]
        neg_sum = acc_ref[2] - pos_sum
        cls_sum = acc_ref[3]

        n_pos_i = n_pos.astype(jnp.int32)
        k = jnp.maximum(n_pos_i, 1)
        n_neg = jnp.maximum(_N_PIX - n_pos_i, 1)

        easy = k >= (_N_PIX - n_pos_i)

        @pl.when(easy)
        def _all():
            acc_ref[4] = neg_sum

        @pl.when(jnp.logical_not(easy))
        def _select():
            def body(i, prefix):
                cand = prefix | (jnp.int32(1) << (30 - i))
                cnt = jnp.sum((bits_ref[...] >= cand).astype(jnp.int32))
                return jnp.where(cnt >= k, cand, prefix)

            prefix = jax.lax.fori_loop(0, 31, body, jnp.int32(0))

            bits = bits_ref[...]
            vals = jax.lax.bitcast_convert_type(bits, jnp.float32)
            gt = bits > prefix
            cnt_gt = jnp.sum(gt.astype(jnp.int32))
            sum_gt = jnp.sum(jnp.where(gt, vals, 0.0))
            tstar = jax.lax.bitcast_convert_type(prefix, jnp.float32)
            acc_ref[4] = sum_gt + (k - cnt_gt).astype(jnp.float32) * tstar

        kworst = acc_ref[4]

        kf = k.astype(jnp.float32)
        nf = n_neg.astype(jnp.float32)
        detection = (_W_POS * pos_sum / kf
                     + _W_NEG * neg_sum / nf
                     + _W_KWORST * kworst / kf)
        out_ref[0, 0] = detection + cls_sum / kf


@functools.partial(jax.jit, static_argnames=("interpret",))
def kernel(y_pred, y, interpret=False):
    out = pl.pallas_call(
        _loss_kernel,
        grid=(_GRID,),
        in_specs=[
            pl.BlockSpec((1, _C, _ROWS_PER_STEP, _W),
                         lambda i: (i // _STEPS_PER_BATCH, 0,
                                    i % _STEPS_PER_BATCH, 0)),
            pl.BlockSpec((1, _C, _ROWS_PER_STEP, _W),
                         lambda i: (i // _STEPS_PER_BATCH, 0,
                                    i % _STEPS_PER_BATCH, 0)),
        ],
        out_specs=pl.BlockSpec(memory_space=pltpu.SMEM),
        out_shape=jax.ShapeDtypeStruct((1, 1), jnp.float32),
        scratch_shapes=[
            pltpu.VMEM((_SCR_ROWS, _W), jnp.int32),
            pltpu.SMEM((8,), jnp.float32),
        ],
        interpret=interpret,
    )(y_pred, y)
    return out[0, 0]
